# baseline (speedup 1.0000x reference)
import functools

import jax
import jax.numpy as jnp
from jax import lax
from jax.experimental import pallas as pl
from jax.experimental.pallas import tpu as pltpu
from jax.experimental.pallas import tpu_sc as plsc

EPS_RMS = 1.1920928955078125e-07
T0 = 16


def _score_select_body(xp_ref, pw_ref, idx_ref, xg_ref):
    xp = xp_ref[0]
    np_, d = xp.shape
    pw = pw_ref[...]
    c = xg_ref.shape[-1]
    xg_ref[...] = xp[:, 12 * c:].reshape(xg_ref.shape)

    ms = jnp.sum(xp * xp, axis=1, keepdims=True) / d
    r = jax.lax.rsqrt(ms + EPS_RMS)
    logits = jnp.sum((xp * r) * pw, axis=1, keepdims=True)

    m = jnp.max(logits, axis=0, keepdims=True)
    e = jnp.exp(logits - m)
    y_col = e / jnp.sum(e, axis=0, keepdims=True)

    pc = jax.lax.broadcasted_iota(jnp.int32, (np_, np_), 0)
    jr = jax.lax.broadcasted_iota(jnp.int32, (np_, np_), 1)
    eye = pc == jr
    y_row = jnp.sum(jnp.where(eye, y_col, 0.0), axis=0, keepdims=True)

    ygt = y_row > y_col
    yeq = y_row == y_col
    ylt = y_row < y_col

    def _ranks(beaten_by, beats):
        rank_col = jnp.sum(beaten_by.astype(jnp.int32), axis=1, keepdims=True)
        rank_row = jnp.sum(beats.astype(jnp.int32), axis=0, keepdims=True)
        return rank_col, rank_row

    top_col, top_row = _ranks(ygt | (yeq & (jr < pc)), ylt | (yeq & (pc < jr)))
    bot_col, bot_row = _ranks(ylt | (yeq & (jr < pc)), ygt | (yeq & (pc < jr)))

    sel_col, sel_row = top_col < 4, top_row < 4
    sel0_col, sel0_row = bot_col < 12, bot_row < 12
    slot_col = jnp.sum(((jr < pc) & sel_row).astype(jnp.int32), axis=1,
                       keepdims=True)
    slot0_col = jnp.sum(((jr < pc) & sel0_row).astype(jnp.int32), axis=1,
                        keepdims=True)

    i16 = jax.lax.broadcasted_iota(jnp.int32, (np_, 16), 1)
    p16 = jax.lax.broadcasted_iota(jnp.int32, (np_, 16), 0)
    top_m = sel_col & (slot_col == i16) & (i16 < 4)
    bot_m = sel0_col & (slot0_col == i16 - 4) & (i16 >= 4)
    idx_vals = jnp.sum(jnp.where(top_m | bot_m, p16, 0), axis=0,
                       keepdims=True)
    idx_ref[...] = idx_vals[None]


def _sc_gather(pidx, x2d, b, t, np_):
    del np_
    c = x2d.shape[1]
    n_rows = b * 128
    rows_per_w = n_rows // 32
    mesh = plsc.VectorSubcoreMesh(core_axis_name="c", subcore_axis_name="s")

    rr = jnp.arange(128, dtype=jnp.int32)
    bi = jnp.arange(b, dtype=jnp.int32)[:, None]
    slot_in_b = jnp.where(rr < 48, rr // 12,
                          jnp.where(rr < 96, 4 + (rr - 48) // 4, 0))
    off = jnp.where(rr < 48, rr % 12, jnp.where(rr < 96, (rr - 48) % 4, 0))
    slotsel = (bi * 16 + slot_in_b[None, :]).reshape(n_rows)
    addbase = jnp.where(rr[None, :] < 96, bi * t + off[None, :],
                        0).reshape(n_rows)

    @functools.partial(
        pl.kernel,
        mesh=mesh,
        out_type=jax.ShapeDtypeStruct((n_rows, c), jnp.float32),
        compiler_params=pltpu.CompilerParams(needs_layout_passes=False),
        scratch_types=[
            pltpu.VMEM((b * 16,), jnp.int32),
            pltpu.VMEM((rows_per_w,), jnp.int32),
            pltpu.VMEM((rows_per_w,), jnp.int32),
            pltpu.VMEM((rows_per_w,), jnp.int32),
            pltpu.VMEM((rows_per_w, c), jnp.float32),
            pltpu.SemaphoreType.DMA,
        ],
    )
    def _gather(slot_hbm, base_hbm, pidx_hbm, x_hbm, out_hbm,
                pv, slotv, basev, idxv, rowsv, sem):
        wid = lax.axis_index("s") * 2 + lax.axis_index("c")
        base = wid * rows_per_w
        pltpu.sync_copy(pidx_hbm, pv)
        pltpu.sync_copy(slot_hbm.at[pl.ds(base, rows_per_w)], slotv)
        pltpu.sync_copy(base_hbm.at[pl.ds(base, rows_per_w)], basev)
        patch = plsc.load_gather(pv, [slotv[...]])
        idxv[...] = basev[...] + patch * T0
        pltpu.async_copy(x_hbm.at[idxv], rowsv, sem).wait()
        pltpu.sync_copy(rowsv, out_hbm.at[pl.ds(base, rows_per_w)])

    return _gather(slotsel, addbase, pidx, x2d)


def _mm4d_body(a_ref, wt_ref, out_ref):
    m = jnp.dot(a_ref[...], wt_ref[...], preferred_element_type=jnp.float32)
    out_ref[...] = m.reshape(out_ref.shape)


def _score_select(xp, patch_w):
    b, np_, d = xp.shape
    c = d // T0
    return pl.pallas_call(
        _score_select_body,
        grid=(b,),
        in_specs=[
            pl.BlockSpec((1, np_, d), lambda i: (i, 0, 0)),
            pl.BlockSpec((1, d), lambda i: (0, 0)),
        ],
        out_specs=[
            pl.BlockSpec((1, 1, 16), lambda i: (i, 0, 0)),
            pl.BlockSpec((4 * np_, c), lambda i: (i, 0)),
        ],
        out_shape=[
            jax.ShapeDtypeStruct((b, 1, 16), jnp.int32),
            jax.ShapeDtypeStruct((b * 4 * np_, c), jnp.float32),
        ],
    )(xp, patch_w.reshape(1, d))


def _project_g(a, wt, b, t, h, c, bm=1024):
    m = a.shape[0]
    per_b = (m // b) // bm
    return pl.pallas_call(
        _mm4d_body,
        grid=(m // bm,),
        in_specs=[
            pl.BlockSpec((bm, c), lambda i: (i, 0)),
            pl.BlockSpec((c, 4 * h * c), lambda i: (0, 0)),
        ],
        out_specs=pl.BlockSpec((1, 4 * bm, h, c),
                               lambda i: (i // per_b, i % per_b, 0, 0)),
        out_shape=jax.ShapeDtypeStruct((b, t, h, c), jnp.float32),
        compiler_params=pltpu.CompilerParams(
            dimension_semantics=("parallel",)),
    )(a, wt)


def _project_sel(a, wt, h, c, bm=256):
    m = a.shape[0]
    return pl.pallas_call(
        _mm4d_body,
        grid=(m // bm,),
        in_specs=[
            pl.BlockSpec((bm, c), lambda i: (i, 0)),
            pl.BlockSpec((c, 4 * h * c), lambda i: (0, 0)),
        ],
        out_specs=pl.BlockSpec((bm, 4, h, c), lambda i: (i, 0, 0, 0)),
        out_shape=jax.ShapeDtypeStruct((m, 4, h, c), jnp.float32),
        compiler_params=pltpu.CompilerParams(
            dimension_semantics=("parallel",)),
    )(a, wt)


def kernel(x, cos, sin, W_qkvg, patch_w, sink):
    del cos, sin
    b, t, c = x.shape
    h = sink.shape[1]
    f = W_qkvg.shape[0]
    np_ = t // T0

    xp = x.reshape(b, np_, T0 * c)
    idx, xg = _score_select(xp, patch_w)
    idx = idx.reshape(b * 16)

    wt = W_qkvg.T
    x2d = x.reshape(b * t, c)

    sel_rows = _sc_gather(idx, x2d, b, t, np_)
    sel4 = _project_sel(sel_rows, wt, h, c).reshape(b, 128, 4, h, c)

    g = _project_g(xg, wt, b, t, h, c)

    qkv = sel4[:, 0:48].reshape(b, 4, 12, 4, h, c)
    q = qkv[:, :, 0:4].reshape(b, 64, h, c)
    k = qkv[:, :, 4:8].reshape(b, 64, h, c)
    v = qkv[:, :, 8:12].reshape(b, 64, h, c)
    q0 = sel4[:, 48:96].reshape(b, 192, h, c)

    s = jnp.broadcast_to(sink, (b, h, c))[:, None, :, :]
    q = jnp.concatenate([s, q], axis=1)
    k = jnp.concatenate([s, k], axis=1)
    v = jnp.concatenate([s, v], axis=1)
    return (q, k, v, g, q0)


# speedup vs baseline: 1.1123x; 1.1123x over previous
import functools

import jax
import jax.numpy as jnp
from jax import lax
from jax.experimental import pallas as pl
from jax.experimental.pallas import tpu as pltpu
from jax.experimental.pallas import tpu_sc as plsc

EPS_RMS = 1.1920928955078125e-07
T0 = 16


def _score_select_body(xp_ref, pw_ref, idx_ref, xg_ref):
    xp = xp_ref[0]
    np_, d = xp.shape
    pw = pw_ref[...]
    c = xg_ref.shape[-1]
    xg_ref[...] = xp[:, 12 * c:].reshape(xg_ref.shape)

    ms = jnp.sum(xp * xp, axis=1, keepdims=True) / d
    r = jax.lax.rsqrt(ms + EPS_RMS)
    logits = jnp.sum((xp * r) * pw, axis=1, keepdims=True)

    m = jnp.max(logits, axis=0, keepdims=True)
    e = jnp.exp(logits - m)
    y_col = e / jnp.sum(e, axis=0, keepdims=True)

    pc = jax.lax.broadcasted_iota(jnp.int32, (np_, np_), 0)
    jr = jax.lax.broadcasted_iota(jnp.int32, (np_, np_), 1)
    eye = pc == jr
    y_row = jnp.sum(jnp.where(eye, y_col, 0.0), axis=0, keepdims=True)

    ygt = y_row > y_col
    yeq = y_row == y_col
    ylt = y_row < y_col

    def _ranks(beaten_by, beats):
        rank_col = jnp.sum(beaten_by.astype(jnp.int32), axis=1, keepdims=True)
        rank_row = jnp.sum(beats.astype(jnp.int32), axis=0, keepdims=True)
        return rank_col, rank_row

    top_col, top_row = _ranks(ygt | (yeq & (jr < pc)), ylt | (yeq & (pc < jr)))
    bot_col, bot_row = _ranks(ylt | (yeq & (jr < pc)), ygt | (yeq & (pc < jr)))

    sel_col, sel_row = top_col < 4, top_row < 4
    sel0_col, sel0_row = bot_col < 12, bot_row < 12
    slot_col = jnp.sum(((jr < pc) & sel_row).astype(jnp.int32), axis=1,
                       keepdims=True)
    slot0_col = jnp.sum(((jr < pc) & sel0_row).astype(jnp.int32), axis=1,
                        keepdims=True)

    i16 = jax.lax.broadcasted_iota(jnp.int32, (np_, 16), 1)
    p16 = jax.lax.broadcasted_iota(jnp.int32, (np_, 16), 0)
    top_m = sel_col & (slot_col == i16) & (i16 < 4)
    bot_m = sel0_col & (slot0_col == i16 - 4) & (i16 >= 4)
    idx_vals = jnp.sum(jnp.where(top_m | bot_m, p16, 0), axis=0,
                       keepdims=True)
    idx_ref[...] = idx_vals[None]


def _sc_gather(pidx, x2d, b, t, np_):
    del np_
    c = x2d.shape[1]
    n_rows = b * 128
    rows_per_w = n_rows // 32
    mesh = plsc.VectorSubcoreMesh(core_axis_name="c", subcore_axis_name="s")

    rr = jnp.arange(128, dtype=jnp.int32)
    bi = jnp.arange(b, dtype=jnp.int32)[:, None]
    slot_in_b = jnp.where(rr < 48, rr // 12,
                          jnp.where(rr < 96, 4 + (rr - 48) // 4, 0))
    off = jnp.where(rr < 48, rr % 12, jnp.where(rr < 96, (rr - 48) % 4, 0))
    slotsel = (bi * 16 + slot_in_b[None, :]).reshape(n_rows)
    addbase = jnp.where(rr[None, :] < 96, bi * t + off[None, :],
                        0).reshape(n_rows)

    @functools.partial(
        pl.kernel,
        mesh=mesh,
        out_type=jax.ShapeDtypeStruct((n_rows, c), jnp.float32),
        compiler_params=pltpu.CompilerParams(needs_layout_passes=False),
        scratch_types=[
            pltpu.VMEM((b * 16,), jnp.int32),
            pltpu.VMEM((rows_per_w,), jnp.int32),
            pltpu.VMEM((rows_per_w,), jnp.int32),
            pltpu.VMEM((rows_per_w,), jnp.int32),
            pltpu.VMEM((rows_per_w, c), jnp.float32),
            pltpu.SemaphoreType.DMA,
        ],
    )
    def _gather(slot_hbm, base_hbm, pidx_hbm, x_hbm, out_hbm,
                pv, slotv, basev, idxv, rowsv, sem):
        wid = lax.axis_index("s") * 2 + lax.axis_index("c")
        base = wid * rows_per_w
        pltpu.sync_copy(pidx_hbm, pv)
        pltpu.sync_copy(slot_hbm.at[pl.ds(base, rows_per_w)], slotv)
        pltpu.sync_copy(base_hbm.at[pl.ds(base, rows_per_w)], basev)
        patch = plsc.load_gather(pv, [slotv[...]])
        idxv[...] = basev[...] + patch * T0
        pltpu.async_copy(x_hbm.at[idxv], rowsv, sem).wait()
        pltpu.sync_copy(rowsv, out_hbm.at[pl.ds(base, rows_per_w)])

    return _gather(slotsel, addbase, pidx, x2d)


def _mm4d_body(a_ref, wt_ref, out_ref):
    m = jnp.dot(a_ref[...], wt_ref[...], preferred_element_type=jnp.float32)
    out_ref[...] = m.reshape(out_ref.shape)


def _score_select(xp, patch_w):
    b, np_, d = xp.shape
    c = d // T0
    return pl.pallas_call(
        _score_select_body,
        grid=(b,),
        in_specs=[
            pl.BlockSpec((1, np_, d), lambda i: (i, 0, 0)),
            pl.BlockSpec((1, d), lambda i: (0, 0)),
        ],
        out_specs=[
            pl.BlockSpec((1, 1, 16), lambda i: (i, 0, 0)),
            pl.BlockSpec((4 * np_, c), lambda i: (i, 0)),
        ],
        out_shape=[
            jax.ShapeDtypeStruct((b, 1, 16), jnp.int32),
            jax.ShapeDtypeStruct((b * 4 * np_, c), jnp.float32),
        ],
    )(xp, patch_w.reshape(1, d))


def _project_g(a, wt, b, t, h, c, bm=512):
    m = a.shape[0]
    per_b = (m // b) // bm
    return pl.pallas_call(
        _mm4d_body,
        grid=(m // bm,),
        in_specs=[
            pl.BlockSpec((bm, c), lambda i: (i, 0)),
            pl.BlockSpec((c, 4 * h * c), lambda i: (0, 0)),
        ],
        out_specs=pl.BlockSpec((1, 4 * bm, h, c),
                               lambda i: (i // per_b, i % per_b, 0, 0)),
        out_shape=jax.ShapeDtypeStruct((b, t, h, c), jnp.float32),
        compiler_params=pltpu.CompilerParams(
            dimension_semantics=("parallel",)),
    )(a, wt)


def _sel_body(a_ref, wt_ref, sink_ref, q_ref, k_ref, v_ref, q0_ref):
    m = jnp.dot(a_ref[...], wt_ref[...], preferred_element_type=jnp.float32)
    h, c = q_ref.shape[-2:]
    m4 = m.reshape(m.shape[0], 4, h, c)
    qkv = m4[0:48].reshape(4, 12, 4, h, c)
    s = sink_ref[...]
    q_ref[0, 0:1] = s
    k_ref[0, 0:1] = s
    v_ref[0, 0:1] = s
    q_ref[0, 1:65] = qkv[:, 0:4].reshape(64, h, c)
    k_ref[0, 1:65] = qkv[:, 4:8].reshape(64, h, c)
    v_ref[0, 1:65] = qkv[:, 8:12].reshape(64, h, c)
    q0_ref[...] = m4[48:96].reshape(1, 192, h, c)


def _project_sel(a, wt, sink, b, h, c):
    qs = jax.ShapeDtypeStruct((b, 65, h, c), jnp.float32)
    return pl.pallas_call(
        _sel_body,
        grid=(b,),
        in_specs=[
            pl.BlockSpec((128, c), lambda i: (i, 0)),
            pl.BlockSpec((c, 4 * h * c), lambda i: (0, 0)),
            pl.BlockSpec((1, h, c), lambda i: (0, 0, 0)),
        ],
        out_specs=[
            pl.BlockSpec((1, 65, h, c), lambda i: (i, 0, 0, 0)),
            pl.BlockSpec((1, 65, h, c), lambda i: (i, 0, 0, 0)),
            pl.BlockSpec((1, 65, h, c), lambda i: (i, 0, 0, 0)),
            pl.BlockSpec((1, 192, h, c), lambda i: (i, 0, 0, 0)),
        ],
        out_shape=[qs, qs, qs,
                   jax.ShapeDtypeStruct((b, 192, h, c), jnp.float32)],
        compiler_params=pltpu.CompilerParams(
            dimension_semantics=("parallel",)),
    )(a, wt, sink)


def kernel(x, cos, sin, W_qkvg, patch_w, sink):
    del cos, sin
    b, t, c = x.shape
    h = sink.shape[1]
    f = W_qkvg.shape[0]
    np_ = t // T0

    xp = x.reshape(b, np_, T0 * c)
    idx, xg = _score_select(xp, patch_w)
    idx = idx.reshape(b * 16)

    wt = W_qkvg.T
    x2d = x.reshape(b * t, c)

    sel_rows = _sc_gather(idx, x2d, b, t, np_)
    q, k, v, q0 = _project_sel(sel_rows, wt, sink, b, h, c)

    g = _project_g(xg, wt, b, t, h, c)

    return (q, k, v, g, q0)


# speedup vs baseline: 1.1156x; 1.0029x over previous
import functools

import jax
import jax.numpy as jnp
from jax import lax
from jax.experimental import pallas as pl
from jax.experimental.pallas import tpu as pltpu
from jax.experimental.pallas import tpu_sc as plsc

EPS_RMS = 1.1920928955078125e-07
T0 = 16


def _score_select_body(xp_ref, pw_ref, idx_ref, xg_ref):
    xp = xp_ref[0]
    np_, d = xp.shape
    pw = pw_ref[...]
    c = xg_ref.shape[-1]
    xg_ref[...] = xp[:, 12 * c:].reshape(xg_ref.shape)

    ms = jnp.sum(xp * xp, axis=1, keepdims=True) / d
    r = jax.lax.rsqrt(ms + EPS_RMS)
    logits = jnp.sum((xp * r) * pw, axis=1, keepdims=True)

    m = jnp.max(logits, axis=0, keepdims=True)
    e = jnp.exp(logits - m)
    y_col = e / jnp.sum(e, axis=0, keepdims=True)

    pc = jax.lax.broadcasted_iota(jnp.int32, (np_, np_), 0)
    jr = jax.lax.broadcasted_iota(jnp.int32, (np_, np_), 1)
    eye = pc == jr
    y_row = jnp.sum(jnp.where(eye, y_col, 0.0), axis=0, keepdims=True)

    ygt = y_row > y_col
    yeq = y_row == y_col
    ylt = y_row < y_col

    def _ranks(beaten_by, beats):
        rank_col = jnp.sum(beaten_by.astype(jnp.int32), axis=1, keepdims=True)
        rank_row = jnp.sum(beats.astype(jnp.int32), axis=0, keepdims=True)
        return rank_col, rank_row

    top_col, top_row = _ranks(ygt | (yeq & (jr < pc)), ylt | (yeq & (pc < jr)))
    bot_col, bot_row = _ranks(ylt | (yeq & (jr < pc)), ygt | (yeq & (pc < jr)))

    sel_col, sel_row = top_col < 4, top_row < 4
    sel0_col, sel0_row = bot_col < 12, bot_row < 12
    slot_col = jnp.sum(((jr < pc) & sel_row).astype(jnp.int32), axis=1,
                       keepdims=True)
    slot0_col = jnp.sum(((jr < pc) & sel0_row).astype(jnp.int32), axis=1,
                        keepdims=True)

    i16 = jax.lax.broadcasted_iota(jnp.int32, (np_, 16), 1)
    p16 = jax.lax.broadcasted_iota(jnp.int32, (np_, 16), 0)
    top_m = sel_col & (slot_col == i16) & (i16 < 4)
    bot_m = sel0_col & (slot0_col == i16 - 4) & (i16 >= 4)
    idx_vals = jnp.sum(jnp.where(top_m | bot_m, p16, 0), axis=0,
                       keepdims=True)
    idx_ref[...] = idx_vals[None]


def _sc_gather(pidx, x2d, b, t, np_):
    del np_
    c = x2d.shape[1]
    n_rows = b * 128
    rows_per_w = n_rows // 32
    mesh = plsc.VectorSubcoreMesh(core_axis_name="c", subcore_axis_name="s")

    rr = jnp.arange(128, dtype=jnp.int32)
    bi = jnp.arange(b, dtype=jnp.int32)[:, None]
    w16 = rr % 16
    slot_in_b = jnp.where(rr < 48, w16 // 4,
                          jnp.where(rr < 96, 4 + (rr - 48) // 4, 0))
    off = jnp.where(rr < 48, (rr // 16) * 4 + w16 % 4,
                    jnp.where(rr < 96, (rr - 48) % 4, 0))
    slotsel = (bi * 16 + slot_in_b[None, :]).reshape(n_rows)
    addbase = jnp.where(rr[None, :] < 96, bi * t + off[None, :],
                        0).reshape(n_rows)

    @functools.partial(
        pl.kernel,
        mesh=mesh,
        out_type=jax.ShapeDtypeStruct((n_rows, c), jnp.float32),
        compiler_params=pltpu.CompilerParams(needs_layout_passes=False),
        scratch_types=[
            pltpu.VMEM((b * 16,), jnp.int32),
            pltpu.VMEM((rows_per_w,), jnp.int32),
            pltpu.VMEM((rows_per_w,), jnp.int32),
            pltpu.VMEM((rows_per_w,), jnp.int32),
            pltpu.VMEM((rows_per_w, c), jnp.float32),
            pltpu.SemaphoreType.DMA,
        ],
    )
    def _gather(slot_hbm, base_hbm, pidx_hbm, x_hbm, out_hbm,
                pv, slotv, basev, idxv, rowsv, sem):
        wid = lax.axis_index("s") * 2 + lax.axis_index("c")
        base = wid * rows_per_w
        pltpu.sync_copy(pidx_hbm, pv)
        pltpu.sync_copy(slot_hbm.at[pl.ds(base, rows_per_w)], slotv)
        pltpu.sync_copy(base_hbm.at[pl.ds(base, rows_per_w)], basev)
        patch = plsc.load_gather(pv, [slotv[...]])
        idxv[...] = basev[...] + patch * T0
        pltpu.async_copy(x_hbm.at[idxv], rowsv, sem).wait()
        pltpu.sync_copy(rowsv, out_hbm.at[pl.ds(base, rows_per_w)])

    return _gather(slotsel, addbase, pidx, x2d)


def _mm4d_body(a_ref, wt_ref, out_ref):
    m = jnp.dot(a_ref[...], wt_ref[...], preferred_element_type=jnp.float32)
    out_ref[...] = m.reshape(out_ref.shape)


def _score_select(xp, patch_w):
    b, np_, d = xp.shape
    c = d // T0
    return pl.pallas_call(
        _score_select_body,
        grid=(b,),
        in_specs=[
            pl.BlockSpec((1, np_, d), lambda i: (i, 0, 0)),
            pl.BlockSpec((1, d), lambda i: (0, 0)),
        ],
        out_specs=[
            pl.BlockSpec((1, 1, 16), lambda i: (i, 0, 0)),
            pl.BlockSpec((4 * np_, c), lambda i: (i, 0)),
        ],
        out_shape=[
            jax.ShapeDtypeStruct((b, 1, 16), jnp.int32),
            jax.ShapeDtypeStruct((b * 4 * np_, c), jnp.float32),
        ],
    )(xp, patch_w.reshape(1, d))


def _project_g(a, wt, b, t, h, c, bm=512):
    m = a.shape[0]
    per_b = (m // b) // bm
    return pl.pallas_call(
        _mm4d_body,
        grid=(m // bm,),
        in_specs=[
            pl.BlockSpec((bm, c), lambda i: (i, 0)),
            pl.BlockSpec((c, 4 * h * c), lambda i: (0, 0)),
        ],
        out_specs=pl.BlockSpec((1, 4 * bm, h, c),
                               lambda i: (i // per_b, i % per_b, 0, 0)),
        out_shape=jax.ShapeDtypeStruct((b, t, h, c), jnp.float32),
        compiler_params=pltpu.CompilerParams(
            dimension_semantics=("parallel",)),
    )(a, wt)


def _sel_body(a_ref, wt_ref, sink_ref, q_ref, k_ref, v_ref, q0_ref):
    m = jnp.dot(a_ref[...], wt_ref[...], preferred_element_type=jnp.float32)
    h, c = q_ref.shape[-2:]
    m4 = m.reshape(m.shape[0], 4, h, c)
    s = sink_ref[...]
    q_ref[0, 0:1] = s
    k_ref[0, 0:1] = s
    v_ref[0, 0:1] = s
    q_ref[0, 1:65] = m4[0:16].reshape(64, h, c)
    k_ref[0, 1:65] = m4[16:32].reshape(64, h, c)
    v_ref[0, 1:65] = m4[32:48].reshape(64, h, c)
    q0_ref[...] = m4[48:96].reshape(1, 192, h, c)


def _project_sel(a, wt, sink, b, h, c):
    qs = jax.ShapeDtypeStruct((b, 65, h, c), jnp.float32)
    return pl.pallas_call(
        _sel_body,
        grid=(b,),
        in_specs=[
            pl.BlockSpec((128, c), lambda i: (i, 0)),
            pl.BlockSpec((c, 4 * h * c), lambda i: (0, 0)),
            pl.BlockSpec((1, h, c), lambda i: (0, 0, 0)),
        ],
        out_specs=[
            pl.BlockSpec((1, 65, h, c), lambda i: (i, 0, 0, 0)),
            pl.BlockSpec((1, 65, h, c), lambda i: (i, 0, 0, 0)),
            pl.BlockSpec((1, 65, h, c), lambda i: (i, 0, 0, 0)),
            pl.BlockSpec((1, 192, h, c), lambda i: (i, 0, 0, 0)),
        ],
        out_shape=[qs, qs, qs,
                   jax.ShapeDtypeStruct((b, 192, h, c), jnp.float32)],
        compiler_params=pltpu.CompilerParams(
            dimension_semantics=("parallel",)),
    )(a, wt, sink)


def kernel(x, cos, sin, W_qkvg, patch_w, sink):
    del cos, sin
    b, t, c = x.shape
    h = sink.shape[1]
    f = W_qkvg.shape[0]
    np_ = t // T0

    xp = x.reshape(b, np_, T0 * c)
    idx, xg = _score_select(xp, patch_w)
    idx = idx.reshape(b * 16)

    wt = W_qkvg.T
    x2d = x.reshape(b * t, c)

    sel_rows = _sc_gather(idx, x2d, b, t, np_)
    q, k, v, q0 = _project_sel(sel_rows, wt, sink, b, h, c)

    g = _project_g(xg, wt, b, t, h, c)

    return (q, k, v, g, q0)


# speedup vs baseline: 1.2289x; 1.1016x over previous
import functools

import jax
import jax.numpy as jnp
from jax import lax
from jax.experimental import pallas as pl
from jax.experimental.pallas import tpu as pltpu
from jax.experimental.pallas import tpu_sc as plsc

EPS_RMS = 1.1920928955078125e-07
T0 = 16


def _score_select_body(x_ref, pw_ref, idx_ref, xg_ref):
    xb = x_ref[0]
    t, c = xb.shape
    np_ = t // T0
    x3 = xb.reshape(np_, T0, c)
    xg_ref[...] = x3[:, 12:16, :].reshape(4 * np_, c)
    pw3 = pw_ref[...][None]

    sq = jnp.sum(x3 * x3, axis=2)
    ms = jnp.sum(sq, axis=1, keepdims=True) / (T0 * c)
    r = jax.lax.rsqrt(ms + EPS_RMS)
    dt = jnp.sum((x3 * r[:, :, None]) * pw3, axis=2)
    logits = jnp.sum(dt, axis=1, keepdims=True)

    m = jnp.max(logits, axis=0, keepdims=True)
    e = jnp.exp(logits - m)
    y_col = e / jnp.sum(e, axis=0, keepdims=True)

    pc = jax.lax.broadcasted_iota(jnp.int32, (np_, np_), 0)
    jr = jax.lax.broadcasted_iota(jnp.int32, (np_, np_), 1)
    eye = pc == jr
    y_row = jnp.sum(jnp.where(eye, y_col, 0.0), axis=0, keepdims=True)

    ygt = y_row > y_col
    yeq = y_row == y_col
    ylt = y_row < y_col

    def _ranks(beaten_by, beats):
        rank_col = jnp.sum(beaten_by.astype(jnp.int32), axis=1, keepdims=True)
        rank_row = jnp.sum(beats.astype(jnp.int32), axis=0, keepdims=True)
        return rank_col, rank_row

    top_col, top_row = _ranks(ygt | (yeq & (jr < pc)), ylt | (yeq & (pc < jr)))
    bot_col, bot_row = _ranks(ylt | (yeq & (jr < pc)), ygt | (yeq & (pc < jr)))

    sel_col, sel_row = top_col < 4, top_row < 4
    sel0_col, sel0_row = bot_col < 12, bot_row < 12
    slot_col = jnp.sum(((jr < pc) & sel_row).astype(jnp.int32), axis=1,
                       keepdims=True)
    slot0_col = jnp.sum(((jr < pc) & sel0_row).astype(jnp.int32), axis=1,
                        keepdims=True)

    i16 = jax.lax.broadcasted_iota(jnp.int32, (np_, 16), 1)
    p16 = jax.lax.broadcasted_iota(jnp.int32, (np_, 16), 0)
    top_m = sel_col & (slot_col == i16) & (i16 < 4)
    bot_m = sel0_col & (slot0_col == i16 - 4) & (i16 >= 4)
    idx_vals = jnp.sum(jnp.where(top_m | bot_m, p16, 0), axis=0,
                       keepdims=True)
    idx_ref[...] = idx_vals[None]


def _sc_gather(pidx, x2d, b, t, np_):
    del np_
    c = x2d.shape[1]
    n_rows = b * 128
    rows_per_w = n_rows // 32
    mesh = plsc.VectorSubcoreMesh(core_axis_name="c", subcore_axis_name="s")

    rr = jnp.arange(128, dtype=jnp.int32)
    bi = jnp.arange(b, dtype=jnp.int32)[:, None]
    w16 = rr % 16
    slot_in_b = jnp.where(rr < 48, w16 // 4,
                          jnp.where(rr < 96, 4 + (rr - 48) // 4, 0))
    off = jnp.where(rr < 48, (rr // 16) * 4 + w16 % 4,
                    jnp.where(rr < 96, (rr - 48) % 4, 0))
    slotsel = (bi * 16 + slot_in_b[None, :]).reshape(n_rows)
    addbase = jnp.where(rr[None, :] < 96, bi * t + off[None, :],
                        0).reshape(n_rows)

    @functools.partial(
        pl.kernel,
        mesh=mesh,
        out_type=jax.ShapeDtypeStruct((n_rows, c), jnp.float32),
        compiler_params=pltpu.CompilerParams(needs_layout_passes=False),
        scratch_types=[
            pltpu.VMEM((b * 16,), jnp.int32),
            pltpu.VMEM((rows_per_w,), jnp.int32),
            pltpu.VMEM((rows_per_w,), jnp.int32),
            pltpu.VMEM((rows_per_w,), jnp.int32),
            pltpu.VMEM((rows_per_w, c), jnp.float32),
            pltpu.SemaphoreType.DMA,
        ],
    )
    def _gather(slot_hbm, base_hbm, pidx_hbm, x_hbm, out_hbm,
                pv, slotv, basev, idxv, rowsv, sem):
        wid = lax.axis_index("s") * 2 + lax.axis_index("c")
        base = wid * rows_per_w
        pltpu.sync_copy(pidx_hbm, pv)
        pltpu.sync_copy(slot_hbm.at[pl.ds(base, rows_per_w)], slotv)
        pltpu.sync_copy(base_hbm.at[pl.ds(base, rows_per_w)], basev)
        patch = plsc.load_gather(pv, [slotv[...]])
        idxv[...] = basev[...] + patch * T0
        pltpu.async_copy(x_hbm.at[idxv], rowsv, sem).wait()
        pltpu.sync_copy(rowsv, out_hbm.at[pl.ds(base, rows_per_w)])

    return _gather(slotsel, addbase, pidx, x2d)


def _mm4d_body(a_ref, wt_ref, out_ref):
    m = jnp.dot(a_ref[...], wt_ref[...], preferred_element_type=jnp.float32)
    out_ref[...] = m.reshape(out_ref.shape)


def _score_select(x, patch_w):
    b, t, c = x.shape
    np_ = t // T0
    return pl.pallas_call(
        _score_select_body,
        grid=(b,),
        in_specs=[
            pl.BlockSpec((1, t, c), lambda i: (i, 0, 0)),
            pl.BlockSpec((T0, c), lambda i: (0, 0)),
        ],
        out_specs=[
            pl.BlockSpec((1, 1, 16), lambda i: (i, 0, 0)),
            pl.BlockSpec((4 * np_, c), lambda i: (i, 0)),
        ],
        out_shape=[
            jax.ShapeDtypeStruct((b, 1, 16), jnp.int32),
            jax.ShapeDtypeStruct((b * 4 * np_, c), jnp.float32),
        ],
    )(x, patch_w.reshape(T0, c))


def _project_g(a, wt, b, t, h, c, bm=512):
    m = a.shape[0]
    per_b = (m // b) // bm
    return pl.pallas_call(
        _mm4d_body,
        grid=(m // bm,),
        in_specs=[
            pl.BlockSpec((bm, c), lambda i: (i, 0)),
            pl.BlockSpec((c, 4 * h * c), lambda i: (0, 0)),
        ],
        out_specs=pl.BlockSpec((1, 4 * bm, h, c),
                               lambda i: (i // per_b, i % per_b, 0, 0)),
        out_shape=jax.ShapeDtypeStruct((b, t, h, c), jnp.float32),
        compiler_params=pltpu.CompilerParams(
            dimension_semantics=("parallel",)),
    )(a, wt)


def _sel_body(a_ref, wt_ref, sink_ref, q_ref, k_ref, v_ref, q0_ref):
    m = jnp.dot(a_ref[...], wt_ref[...], preferred_element_type=jnp.float32)
    h, c = q_ref.shape[-2:]
    m4 = m.reshape(m.shape[0], 4, h, c)
    s = sink_ref[...]
    q_ref[0, 0:1] = s
    k_ref[0, 0:1] = s
    v_ref[0, 0:1] = s
    q_ref[0, 1:65] = m4[0:16].reshape(64, h, c)
    k_ref[0, 1:65] = m4[16:32].reshape(64, h, c)
    v_ref[0, 1:65] = m4[32:48].reshape(64, h, c)
    q0_ref[...] = m4[48:96].reshape(1, 192, h, c)


def _project_sel(a, wt, sink, b, h, c):
    qs = jax.ShapeDtypeStruct((b, 65, h, c), jnp.float32)
    return pl.pallas_call(
        _sel_body,
        grid=(b,),
        in_specs=[
            pl.BlockSpec((128, c), lambda i: (i, 0)),
            pl.BlockSpec((c, 4 * h * c), lambda i: (0, 0)),
            pl.BlockSpec((1, h, c), lambda i: (0, 0, 0)),
        ],
        out_specs=[
            pl.BlockSpec((1, 65, h, c), lambda i: (i, 0, 0, 0)),
            pl.BlockSpec((1, 65, h, c), lambda i: (i, 0, 0, 0)),
            pl.BlockSpec((1, 65, h, c), lambda i: (i, 0, 0, 0)),
            pl.BlockSpec((1, 192, h, c), lambda i: (i, 0, 0, 0)),
        ],
        out_shape=[qs, qs, qs,
                   jax.ShapeDtypeStruct((b, 192, h, c), jnp.float32)],
        compiler_params=pltpu.CompilerParams(
            dimension_semantics=("parallel",)),
    )(a, wt, sink)


def kernel(x, cos, sin, W_qkvg, patch_w, sink):
    del cos, sin
    b, t, c = x.shape
    h = sink.shape[1]
    f = W_qkvg.shape[0]
    np_ = t // T0

    idx, xg = _score_select(x, patch_w)
    idx = idx.reshape(b * 16)

    wt = W_qkvg.T
    x2d = x.reshape(b * t, c)

    sel_rows = _sc_gather(idx, x2d, b, t, np_)
    q, k, v, q0 = _project_sel(sel_rows, wt, sink, b, h, c)

    g = _project_g(xg, wt, b, t, h, c)

    return (q, k, v, g, q0)


# speedup vs baseline: 1.3035x; 1.0607x over previous
import functools

import jax
import jax.numpy as jnp
from jax import lax
from jax.experimental import pallas as pl
from jax.experimental.pallas import tpu as pltpu
from jax.experimental.pallas import tpu_sc as plsc

EPS_RMS = 1.1920928955078125e-07
T0 = 16


def _score_select_body(x_ref, pw_ref, idx_ref, xg_ref):
    xb = x_ref[0]
    t, c = xb.shape
    np_ = t // T0
    x3 = xb.reshape(np_, T0, c)
    xg_ref[...] = x3[:, 12:16, :].reshape(4 * np_, c)
    pw3 = pw_ref[...][None]

    sq = jnp.sum(x3 * x3, axis=2)
    ms = jnp.sum(sq, axis=1, keepdims=True) / (T0 * c)
    r = jax.lax.rsqrt(ms + EPS_RMS)
    dt = jnp.sum((x3 * r[:, :, None]) * pw3, axis=2)
    logits = jnp.sum(dt, axis=1, keepdims=True)

    m = jnp.max(logits, axis=0, keepdims=True)
    e = jnp.exp(logits - m)
    y_col = e / jnp.sum(e, axis=0, keepdims=True)

    pc = jax.lax.broadcasted_iota(jnp.int32, (np_, np_), 0)
    jr = jax.lax.broadcasted_iota(jnp.int32, (np_, np_), 1)
    eye = pc == jr
    y_row = jnp.sum(jnp.where(eye, y_col, 0.0), axis=0, keepdims=True)

    ygt = y_row > y_col
    yeq = y_row == y_col
    ylt = y_row < y_col

    def _ranks(beaten_by, beats):
        rank_col = jnp.sum(beaten_by.astype(jnp.int32), axis=1, keepdims=True)
        rank_row = jnp.sum(beats.astype(jnp.int32), axis=0, keepdims=True)
        return rank_col, rank_row

    top_col, top_row = _ranks(ygt | (yeq & (jr < pc)), ylt | (yeq & (pc < jr)))
    bot_col, bot_row = _ranks(ylt | (yeq & (jr < pc)), ygt | (yeq & (pc < jr)))

    sel_col, sel_row = top_col < 4, top_row < 4
    sel0_col, sel0_row = bot_col < 12, bot_row < 12
    slot_col = jnp.sum(((jr < pc) & sel_row).astype(jnp.int32), axis=1,
                       keepdims=True)
    slot0_col = jnp.sum(((jr < pc) & sel0_row).astype(jnp.int32), axis=1,
                        keepdims=True)

    i16 = jax.lax.broadcasted_iota(jnp.int32, (np_, 16), 1)
    p16 = jax.lax.broadcasted_iota(jnp.int32, (np_, 16), 0)
    top_m = sel_col & (slot_col == i16) & (i16 < 4)
    bot_m = sel0_col & (slot0_col == i16 - 4) & (i16 >= 4)
    idx_vals = jnp.sum(jnp.where(top_m | bot_m, p16, 0), axis=0,
                       keepdims=True)
    idx_ref[...] = idx_vals[None]


def _sc_gather(pidx, x2d, b, t, np_):
    del np_
    c = x2d.shape[1]
    n_rows = b * 128
    rows_per_w = n_rows // 32
    mesh = plsc.VectorSubcoreMesh(core_axis_name="c", subcore_axis_name="s")

    rr = jnp.arange(128, dtype=jnp.int32)
    bi = jnp.arange(b, dtype=jnp.int32)[:, None]
    w16 = rr % 16
    slot_in_b = jnp.where(rr < 48, w16 // 4,
                          jnp.where(rr < 96, 4 + (rr - 48) // 4, 0))
    off = jnp.where(rr < 48, (rr // 16) * 4 + w16 % 4,
                    jnp.where(rr < 96, (rr - 48) % 4, 0))
    slotsel = (bi * 16 + slot_in_b[None, :]).reshape(n_rows)
    addbase = jnp.where(rr[None, :] < 96, bi * t + off[None, :],
                        0).reshape(n_rows)

    @functools.partial(
        pl.kernel,
        mesh=mesh,
        out_type=jax.ShapeDtypeStruct((n_rows, c), jnp.float32),
        compiler_params=pltpu.CompilerParams(needs_layout_passes=False),
        scratch_types=[
            pltpu.VMEM((b * 16,), jnp.int32),
            pltpu.VMEM((rows_per_w,), jnp.int32),
            pltpu.VMEM((rows_per_w,), jnp.int32),
            pltpu.VMEM((rows_per_w,), jnp.int32),
            pltpu.VMEM((rows_per_w, c), jnp.float32),
            pltpu.SemaphoreType.DMA,
        ],
    )
    def _gather(slot_hbm, base_hbm, pidx_hbm, x_hbm, out_hbm,
                pv, slotv, basev, idxv, rowsv, sem):
        wid = lax.axis_index("s") * 2 + lax.axis_index("c")
        base = wid * rows_per_w
        pltpu.sync_copy(pidx_hbm, pv)
        pltpu.sync_copy(slot_hbm.at[pl.ds(base, rows_per_w)], slotv)
        pltpu.sync_copy(base_hbm.at[pl.ds(base, rows_per_w)], basev)
        patch = plsc.load_gather(pv, [slotv[...]])
        idxv[...] = basev[...] + patch * T0
        pltpu.async_copy(x_hbm.at[idxv], rowsv, sem).wait()
        pltpu.sync_copy(rowsv, out_hbm.at[pl.ds(base, rows_per_w)])

    return _gather(slotsel, addbase, pidx, x2d)


def _mmt(a, w):
    return lax.dot_general(a, w, (((1,), (1,)), ((), ())),
                           preferred_element_type=jnp.float32)


def _mm4d_body(a_ref, w_ref, out_ref):
    m = _mmt(a_ref[...], w_ref[...])
    out_ref[...] = m.reshape(out_ref.shape)


def _score_select(x, patch_w):
    b, t, c = x.shape
    np_ = t // T0
    return pl.pallas_call(
        _score_select_body,
        grid=(b,),
        in_specs=[
            pl.BlockSpec((1, t, c), lambda i: (i, 0, 0)),
            pl.BlockSpec((T0, c), lambda i: (0, 0)),
        ],
        out_specs=[
            pl.BlockSpec((1, 1, 16), lambda i: (i, 0, 0)),
            pl.BlockSpec((4 * np_, c), lambda i: (i, 0)),
        ],
        out_shape=[
            jax.ShapeDtypeStruct((b, 1, 16), jnp.int32),
            jax.ShapeDtypeStruct((b * 4 * np_, c), jnp.float32),
        ],
    )(x, patch_w.reshape(T0, c))


def _project_g(a, wt, b, t, h, c, bm=512):
    m = a.shape[0]
    per_b = (m // b) // bm
    return pl.pallas_call(
        _mm4d_body,
        grid=(m // bm,),
        in_specs=[
            pl.BlockSpec((bm, c), lambda i: (i, 0)),
            pl.BlockSpec((4 * h * c, c), lambda i: (0, 0)),
        ],
        out_specs=pl.BlockSpec((1, 4 * bm, h, c),
                               lambda i: (i // per_b, i % per_b, 0, 0)),
        out_shape=jax.ShapeDtypeStruct((b, t, h, c), jnp.float32),
        compiler_params=pltpu.CompilerParams(
            dimension_semantics=("parallel",)),
    )(a, wt)


def _sel_body(a_ref, w_ref, sink_ref, q_ref, k_ref, v_ref, q0_ref):
    m = _mmt(a_ref[...], w_ref[...])
    h, c = q_ref.shape[-2:]
    m4 = m.reshape(m.shape[0], 4, h, c)
    s = sink_ref[...]
    q_ref[0, 0:1] = s
    k_ref[0, 0:1] = s
    v_ref[0, 0:1] = s
    q_ref[0, 1:65] = m4[0:16].reshape(64, h, c)
    k_ref[0, 1:65] = m4[16:32].reshape(64, h, c)
    v_ref[0, 1:65] = m4[32:48].reshape(64, h, c)
    q0_ref[...] = m4[48:96].reshape(1, 192, h, c)


def _project_sel(a, wt, sink, b, h, c):
    qs = jax.ShapeDtypeStruct((b, 65, h, c), jnp.float32)
    return pl.pallas_call(
        _sel_body,
        grid=(b,),
        in_specs=[
            pl.BlockSpec((128, c), lambda i: (i, 0)),
            pl.BlockSpec((4 * h * c, c), lambda i: (0, 0)),
            pl.BlockSpec((1, h, c), lambda i: (0, 0, 0)),
        ],
        out_specs=[
            pl.BlockSpec((1, 65, h, c), lambda i: (i, 0, 0, 0)),
            pl.BlockSpec((1, 65, h, c), lambda i: (i, 0, 0, 0)),
            pl.BlockSpec((1, 65, h, c), lambda i: (i, 0, 0, 0)),
            pl.BlockSpec((1, 192, h, c), lambda i: (i, 0, 0, 0)),
        ],
        out_shape=[qs, qs, qs,
                   jax.ShapeDtypeStruct((b, 192, h, c), jnp.float32)],
        compiler_params=pltpu.CompilerParams(
            dimension_semantics=("parallel",)),
    )(a, wt, sink)


def kernel(x, cos, sin, W_qkvg, patch_w, sink):
    del cos, sin
    b, t, c = x.shape
    h = sink.shape[1]
    f = W_qkvg.shape[0]
    np_ = t // T0

    idx, xg = _score_select(x, patch_w)
    idx = idx.reshape(b * 16)

    x2d = x.reshape(b * t, c)

    sel_rows = _sc_gather(idx, x2d, b, t, np_)
    q, k, v, q0 = _project_sel(sel_rows, W_qkvg, sink, b, h, c)

    g = _project_g(xg, W_qkvg, b, t, h, c)

    return (q, k, v, g, q0)
